# baseline (speedup 1.0000x reference)
import functools

import jax
import jax.numpy as jnp
from jax import lax
from jax.experimental import pallas as pl
from jax.experimental.pallas import tpu as pltpu
from jax.experimental.pallas import tpu_sc as plsc

E = 8
K = 2
D = 1024
F = 512
T = 4096
BM = 256
NBLK = T * K // BM + E
NP = NBLK * BM
NW = 32


def _router_body(x_ref, gw_ref, dest_ref, wts_ref, emap_ref):
    x = x_ref[...]
    gw = gw_ref[...]
    logits = lax.dot_general(x, gw, (((1,), (1,)), ((), ())),
                             preferred_element_type=jnp.float32)
    m = jnp.max(logits, axis=1, keepdims=True)
    ex = jnp.exp(logits - m)
    p = ex / jnp.sum(ex, axis=1, keepdims=True)

    iota_e = lax.broadcasted_iota(jnp.int32, (T, E), 1)
    m1 = jnp.max(p, axis=1, keepdims=True)
    i1 = jnp.min(jnp.where(p == m1, iota_e, E), axis=1, keepdims=True)
    pm = jnp.where(iota_e == i1, -jnp.inf, p)
    m2 = jnp.max(pm, axis=1, keepdims=True)
    i2 = jnp.min(jnp.where(pm == m2, iota_e, E), axis=1, keepdims=True)
    s = m1 + m2
    w_cat = jnp.concatenate([m1 / s, m2 / s], axis=0)
    wts_ref[...] = jnp.broadcast_to(w_cat, (K * T, 16))

    ef = jnp.concatenate([i1.reshape(1, T), i2.reshape(1, T)], axis=1)
    rows = lax.broadcasted_iota(jnp.int32, (E, K * T), 0)
    onehot = (rows == ef).astype(jnp.int32)

    c = onehot
    sh = 1
    while sh < K * T:
        z = jnp.zeros((E, sh), jnp.int32)
        c = c + jnp.concatenate([z, c[:, : K * T - sh]], axis=1)
        sh *= 2
    excl = c - onehot
    counts = c[:, K * T - 1 :]
    cnt_pad = ((counts + (BM - 1)) // BM) * BM

    cc = cnt_pad
    for sh in (1, 2, 4):
        z = jnp.zeros((sh, 1), jnp.int32)
        cc = cc + jnp.concatenate([z, cc[: E - sh]], axis=0)
    off = cc - cnt_pad

    rank = jnp.sum(onehot * excl, axis=0, keepdims=True)
    offj = jnp.sum(onehot * jnp.broadcast_to(off, (E, K * T)), axis=0,
                   keepdims=True)
    dest_ref[...] = rank + offj

    bcols = lax.broadcasted_iota(jnp.int32, (E, NBLK), 1) * BM
    cmp = (jnp.broadcast_to(cc, (E, NBLK)) <= bcols).astype(jnp.int32)
    emap = jnp.minimum(jnp.sum(cmp, axis=0), E - 1)
    nreal = cc[E - 1] // BM
    emap_ref[...] = jnp.concatenate([emap, nreal], axis=0)


def _router_call(x, gate_w):
    return pl.pallas_call(
        _router_body,
        out_shape=(
            jax.ShapeDtypeStruct((1, K * T), jnp.int32),
            jax.ShapeDtypeStruct((K * T, 16), jnp.float32),
            jax.ShapeDtypeStruct((NBLK + 1,), jnp.int32),
        ),
    )(x, gate_w)


_CH = 64
_TPS = T // NW
_SNCH = _TPS // _CH


def _scatter_kernel(x_hbm, dest_hbm, xs_hbm,
                    idx0_v, idx1_v, rows_v, sem_out):
    wid = lax.axis_index("s") * 2 + lax.axis_index("c")
    tb = wid * _TPS
    for cidx in range(_SNCH):
        base = pl.multiple_of(tb + cidx * _CH, _CH)
        pltpu.sync_copy(dest_hbm.at[0, pl.ds(base, _CH)], idx0_v.at[cidx])
        pltpu.sync_copy(dest_hbm.at[1, pl.ds(base, _CH)], idx1_v.at[cidx])
        pltpu.sync_copy(x_hbm.at[pl.ds(base, _CH)], rows_v)
        r0 = pltpu.async_copy(rows_v, xs_hbm.at[idx0_v.at[cidx]], sem_out)
        r1 = pltpu.async_copy(rows_v, xs_hbm.at[idx1_v.at[cidx]], sem_out)
        r0.wait()
        r1.wait()


def _scatter_call(x, dest):
    mesh = plsc.VectorSubcoreMesh(core_axis_name="c", subcore_axis_name="s")
    kfn = pl.kernel(
        _scatter_kernel,
        mesh=mesh,
        out_type=jax.ShapeDtypeStruct((NP, D), jnp.float32),
        scratch_types=[
            pltpu.VMEM((_SNCH, _CH), jnp.int32),
            pltpu.VMEM((_SNCH, _CH), jnp.int32),
            pltpu.VMEM((_CH, D), jnp.float32),
            pltpu.SemaphoreType.DMA,
        ],
    )
    return kfn(x, dest)


def _gmm_body(emap_ref, xs_ref, w1_ref, w2_ref, ys_ref):
    b = pl.program_id(0)

    @pl.when(b < emap_ref[NBLK])
    def _():
        x = xs_ref[...]
        w1b = w1_ref[0]
        gu = lax.dot_general(x, w1b, (((1,), (1,)), ((), ())),
                             preferred_element_type=jnp.float32)
        gate = gu[:, :F]
        up = gu[:, F:]
        act = gate * (1.0 / (1.0 + jnp.exp(-gate))) * up
        w2b = w2_ref[0]
        ys_ref[...] = lax.dot_general(act, w2b, (((1,), (1,)), ((), ())),
                                      preferred_element_type=jnp.float32)


def _gmm_call(emap, xs, w1, w2):
    grid_spec = pltpu.PrefetchScalarGridSpec(
        num_scalar_prefetch=1,
        grid=(NBLK,),
        in_specs=[
            pl.BlockSpec((BM, D), lambda b, emap: (b, 0)),
            pl.BlockSpec((1, 2 * F, D), lambda b, emap: (emap[b], 0, 0)),
            pl.BlockSpec((1, D, F), lambda b, emap: (emap[b], 0, 0)),
        ],
        out_specs=pl.BlockSpec((BM, D), lambda b, emap: (b, 0)),
    )
    return pl.pallas_call(
        _gmm_body,
        grid_spec=grid_spec,
        out_shape=jax.ShapeDtypeStruct((NP, D), jnp.float32),
    )(emap, xs, w1, w2)


_CC = 8
_TPW = T // NW


def _combine_kernel(ys_hbm, dest_hbm, wts_hbm, out_hbm,
                    idx0_v, idx1_v, w0_v, w1_v, y0_v, y1_v, out_v,
                    sem_in, sem_out):
    wid = lax.axis_index("s") * 2 + lax.axis_index("c")
    tb = wid * _TPW
    nch = _TPW // _CC
    pltpu.sync_copy(wts_hbm.at[pl.ds(tb, _TPW)], w0_v)
    pltpu.sync_copy(wts_hbm.at[pl.ds(T + tb, _TPW)], w1_v)

    def start_gather(cidx, slot):
        base = pl.multiple_of(tb + cidx * _CC, _CC)
        pltpu.sync_copy(dest_hbm.at[0, pl.ds(base, _CC)], idx0_v.at[cidx])
        pltpu.sync_copy(dest_hbm.at[1, pl.ds(base, _CC)], idx1_v.at[cidx])
        return (pltpu.async_copy(ys_hbm.at[idx0_v.at[cidx]], y0_v.at[slot],
                                 sem_in),
                pltpu.async_copy(ys_hbm.at[idx1_v.at[cidx]], y1_v.at[slot],
                                 sem_in))

    gets = {0: start_gather(0, 0), 1: start_gather(1, 1)}
    puts = {}
    for cidx in range(nch):
        slot = cidx % 3
        if cidx + 2 < nch:
            gets[cidx + 2] = start_gather(cidx + 2, (cidx + 2) % 3)
        g0, g1 = gets.pop(cidx)
        g0.wait()
        g1.wait()
        if cidx >= 2:
            puts.pop(cidx - 2).wait()

        def tok_body(i, _):
            s0 = w0_v[cidx * _CC + i, :]
            s1 = w1_v[cidx * _CC + i, :]

            def vec_body(v, _):
                col = v * 64
                for u in range(4):
                    sl = pl.ds(col + u * 16, 16)
                    out_v[cidx % 2, i, sl] = (s0 * y0_v[slot, i, sl]
                                              + s1 * y1_v[slot, i, sl])
                return 0

            lax.fori_loop(0, D // 64, vec_body, 0)
            return 0

        lax.fori_loop(0, _CC, tok_body, 0)
        base = pl.multiple_of(tb + cidx * _CC, _CC)
        puts[cidx] = pltpu.async_copy(out_v.at[cidx % 2],
                                      out_hbm.at[pl.ds(base, _CC)], sem_out)
    for cp in puts.values():
        cp.wait()


def _combine_call(ys, dest, wts):
    mesh = plsc.VectorSubcoreMesh(core_axis_name="c", subcore_axis_name="s")
    kfn = pl.kernel(
        _combine_kernel,
        mesh=mesh,
        out_type=jax.ShapeDtypeStruct((T, D), jnp.float32),
        scratch_types=[
            pltpu.VMEM((_TPW // _CC, _CC), jnp.int32),
            pltpu.VMEM((_TPW // _CC, _CC), jnp.int32),
            pltpu.VMEM((_TPW, 16), jnp.float32),
            pltpu.VMEM((_TPW, 16), jnp.float32),
            pltpu.VMEM((3, _CC, D), jnp.float32),
            pltpu.VMEM((3, _CC, D), jnp.float32),
            pltpu.VMEM((2, _CC, D), jnp.float32),
            pltpu.SemaphoreType.DMA,
            pltpu.SemaphoreType.DMA,
        ],
    )
    return kfn(ys, dest, wts)


@jax.jit
def kernel(hidden_states, gate_w, w1, w2):
    destf, wts, emap = _router_call(hidden_states, gate_w)
    dest = destf.reshape(K, T)
    xs = _scatter_call(hidden_states, dest)
    ys = _gmm_call(emap, xs, w1, w2)
    return _combine_call(ys, dest, wts)


# speedup vs baseline: 1.2093x; 1.2093x over previous
import functools

import jax
import jax.numpy as jnp
from jax import lax
from jax.experimental import pallas as pl
from jax.experimental.pallas import tpu as pltpu
from jax.experimental.pallas import tpu_sc as plsc

E = 8
K = 2
D = 1024
F = 512
T = 4096
BM = 512
NBLK = T * K // BM + E
NP = NBLK * BM
NW = 32


def _router_body(x_ref, gw_ref, dest_ref, wts_ref, emap_ref):
    x = x_ref[...]
    gw = gw_ref[...]
    logits = lax.dot_general(x, gw, (((1,), (1,)), ((), ())),
                             preferred_element_type=jnp.float32)
    m = jnp.max(logits, axis=1, keepdims=True)
    ex = jnp.exp(logits - m)
    p = ex / jnp.sum(ex, axis=1, keepdims=True)

    iota_e = lax.broadcasted_iota(jnp.int32, (T, E), 1)
    m1 = jnp.max(p, axis=1, keepdims=True)
    i1 = jnp.min(jnp.where(p == m1, iota_e, E), axis=1, keepdims=True)
    pm = jnp.where(iota_e == i1, -jnp.inf, p)
    m2 = jnp.max(pm, axis=1, keepdims=True)
    i2 = jnp.min(jnp.where(pm == m2, iota_e, E), axis=1, keepdims=True)
    s = m1 + m2
    w_cat = jnp.concatenate([m1 / s, m2 / s], axis=0)
    wts_ref[...] = jnp.broadcast_to(w_cat, (K * T, 16))

    ef = jnp.concatenate([i1.reshape(1, T), i2.reshape(1, T)], axis=1)
    rows = lax.broadcasted_iota(jnp.int32, (E, K * T), 0)
    onehot = (rows == ef).astype(jnp.int32)

    c = onehot
    sh = 1
    while sh < K * T:
        z = jnp.zeros((E, sh), jnp.int32)
        c = c + jnp.concatenate([z, c[:, : K * T - sh]], axis=1)
        sh *= 2
    excl = c - onehot
    counts = c[:, K * T - 1 :]
    cnt_pad = ((counts + (BM - 1)) // BM) * BM

    cc = cnt_pad
    for sh in (1, 2, 4):
        z = jnp.zeros((sh, 1), jnp.int32)
        cc = cc + jnp.concatenate([z, cc[: E - sh]], axis=0)
    off = cc - cnt_pad

    rank = jnp.sum(onehot * excl, axis=0, keepdims=True)
    offj = jnp.sum(onehot * jnp.broadcast_to(off, (E, K * T)), axis=0,
                   keepdims=True)
    dest_ref[...] = rank + offj

    bcols = lax.broadcasted_iota(jnp.int32, (E, NBLK), 1) * BM
    bcols = jnp.minimum(bcols, cc[E - 1 :, :] - BM)
    cmp = (jnp.broadcast_to(cc, (E, NBLK)) <= bcols).astype(jnp.int32)
    emap = jnp.minimum(jnp.sum(cmp, axis=0), E - 1)
    nreal = cc[E - 1] // BM
    emap_ref[...] = jnp.concatenate([emap, nreal], axis=0)


def _router_call(x, gate_w):
    return pl.pallas_call(
        _router_body,
        out_shape=(
            jax.ShapeDtypeStruct((1, K * T), jnp.int32),
            jax.ShapeDtypeStruct((K * T, 16), jnp.float32),
            jax.ShapeDtypeStruct((NBLK + 1,), jnp.int32),
        ),
    )(x, gate_w)


_CH = 64
_TPS = T // NW
_SNCH = _TPS // _CH


def _scatter_kernel(x_hbm, dest_hbm, xs_hbm,
                    idx0_v, idx1_v, rows_v, sem_out):
    wid = lax.axis_index("s") * 2 + lax.axis_index("c")
    tb = wid * _TPS
    for cidx in range(_SNCH):
        base = pl.multiple_of(tb + cidx * _CH, _CH)
        pltpu.sync_copy(dest_hbm.at[0, pl.ds(base, _CH)], idx0_v.at[cidx])
        pltpu.sync_copy(dest_hbm.at[1, pl.ds(base, _CH)], idx1_v.at[cidx])
        pltpu.sync_copy(x_hbm.at[pl.ds(base, _CH)], rows_v)
        r0 = pltpu.async_copy(rows_v, xs_hbm.at[idx0_v.at[cidx]], sem_out)
        r1 = pltpu.async_copy(rows_v, xs_hbm.at[idx1_v.at[cidx]], sem_out)
        r0.wait()
        r1.wait()


def _scatter_call(x, dest):
    mesh = plsc.VectorSubcoreMesh(core_axis_name="c", subcore_axis_name="s")
    kfn = pl.kernel(
        _scatter_kernel,
        mesh=mesh,
        out_type=jax.ShapeDtypeStruct((NP, D), jnp.float32),
        scratch_types=[
            pltpu.VMEM((_SNCH, _CH), jnp.int32),
            pltpu.VMEM((_SNCH, _CH), jnp.int32),
            pltpu.VMEM((_CH, D), jnp.float32),
            pltpu.SemaphoreType.DMA,
        ],
    )
    return kfn(x, dest)


def _gmm_body(emap_ref, xs_ref, w1_ref, w2_ref, ys_ref):
    b = pl.program_id(0)

    @pl.when(b < emap_ref[NBLK])
    def _():
        x = xs_ref[...]
        w1b = w1_ref[0]
        gu = lax.dot_general(x, w1b, (((1,), (1,)), ((), ())),
                             preferred_element_type=jnp.float32)
        gate = gu[:, :F]
        up = gu[:, F:]
        act = gate * (1.0 / (1.0 + jnp.exp(-gate))) * up
        w2b = w2_ref[0]
        ys_ref[...] = lax.dot_general(act, w2b, (((1,), (1,)), ((), ())),
                                      preferred_element_type=jnp.float32)


def _gmm_call(emap, xs, w1, w2):
    grid_spec = pltpu.PrefetchScalarGridSpec(
        num_scalar_prefetch=1,
        grid=(NBLK,),
        in_specs=[
            pl.BlockSpec((BM, D),
                         lambda b, emap: (jnp.minimum(b, emap[NBLK] - 1), 0)),
            pl.BlockSpec((1, 2 * F, D), lambda b, emap: (emap[b], 0, 0)),
            pl.BlockSpec((1, D, F), lambda b, emap: (emap[b], 0, 0)),
        ],
        out_specs=pl.BlockSpec((BM, D), lambda b, emap: (b, 0)),
    )
    return pl.pallas_call(
        _gmm_body,
        grid_spec=grid_spec,
        out_shape=jax.ShapeDtypeStruct((NP, D), jnp.float32),
    )(emap, xs, w1, w2)


_CC = 8
_TPW = T // NW


def _combine_kernel(ys_hbm, dest_hbm, wts_hbm, out_hbm,
                    idx0_v, idx1_v, w0_v, w1_v, y0_v, y1_v, out_v,
                    sem_in, sem_out):
    wid = lax.axis_index("s") * 2 + lax.axis_index("c")
    tb = wid * _TPW
    nch = _TPW // _CC
    pltpu.sync_copy(wts_hbm.at[pl.ds(tb, _TPW)], w0_v)
    pltpu.sync_copy(wts_hbm.at[pl.ds(T + tb, _TPW)], w1_v)

    def start_gather(cidx, slot):
        base = pl.multiple_of(tb + cidx * _CC, _CC)
        pltpu.sync_copy(dest_hbm.at[0, pl.ds(base, _CC)], idx0_v.at[cidx])
        pltpu.sync_copy(dest_hbm.at[1, pl.ds(base, _CC)], idx1_v.at[cidx])
        return (pltpu.async_copy(ys_hbm.at[idx0_v.at[cidx]], y0_v.at[slot],
                                 sem_in),
                pltpu.async_copy(ys_hbm.at[idx1_v.at[cidx]], y1_v.at[slot],
                                 sem_in))

    gets = {0: start_gather(0, 0)}
    puts = {}
    for cidx in range(nch):
        slot = cidx % 2
        if cidx + 1 < nch:
            gets[cidx + 1] = start_gather(cidx + 1, (cidx + 1) % 2)
        g0, g1 = gets.pop(cidx)
        g0.wait()
        g1.wait()
        if cidx >= 2:
            puts.pop(cidx - 2).wait()

        def tok_body(i, _):
            s0 = w0_v[cidx * _CC + i, :]
            s1 = w1_v[cidx * _CC + i, :]

            def vec_body(v, _):
                col = v * 64
                for u in range(4):
                    sl = pl.ds(col + u * 16, 16)
                    out_v[slot, i, sl] = (s0 * y0_v[slot, i, sl]
                                          + s1 * y1_v[slot, i, sl])
                return 0

            lax.fori_loop(0, D // 64, vec_body, 0)
            return 0

        lax.fori_loop(0, _CC, tok_body, 0)
        base = pl.multiple_of(tb + cidx * _CC, _CC)
        puts[cidx] = pltpu.async_copy(out_v.at[slot],
                                      out_hbm.at[pl.ds(base, _CC)], sem_out)
    for cp in puts.values():
        cp.wait()


def _combine_call(ys, dest, wts):
    mesh = plsc.VectorSubcoreMesh(core_axis_name="c", subcore_axis_name="s")
    kfn = pl.kernel(
        _combine_kernel,
        mesh=mesh,
        out_type=jax.ShapeDtypeStruct((T, D), jnp.float32),
        scratch_types=[
            pltpu.VMEM((_TPW // _CC, _CC), jnp.int32),
            pltpu.VMEM((_TPW // _CC, _CC), jnp.int32),
            pltpu.VMEM((_TPW, 16), jnp.float32),
            pltpu.VMEM((_TPW, 16), jnp.float32),
            pltpu.VMEM((2, _CC, D), jnp.float32),
            pltpu.VMEM((2, _CC, D), jnp.float32),
            pltpu.VMEM((2, _CC, D), jnp.float32),
            pltpu.SemaphoreType.DMA,
            pltpu.SemaphoreType.DMA,
        ],
    )
    return kfn(ys, dest, wts)


@jax.jit
def kernel(hidden_states, gate_w, w1, w2):
    destf, wts, emap = _router_call(hidden_states, gate_w)
    dest = destf.reshape(K, T)
    xs = _scatter_call(hidden_states, dest)
    ys = _gmm_call(emap, xs, w1, w2)
    return _combine_call(ys, dest, wts)
